# baseline (speedup 1.0000x reference)
import functools

import jax
import jax.numpy as jnp
from jax import lax
from jax.experimental import pallas as pl
from jax.experimental.pallas import tpu as pltpu
from jax.experimental.pallas import tpu_sc as plsc

_EMBED = 32
_BATCH = 16384
_ROWS = 1000000
_NC = 2
_NS = 16
_NW = _NC * _NS
_BPW = _BATCH // _NW
_CHUNK = 512

_SEG = 4096
_SH = 12
_G = 31
_PACKED_ROWS = _G * _SEG
_NBLK = (_ROWS + _SEG - 1) // _SEG - 1


def _repack_body(ea_ref, eb_ref, u0, u1, u2, u3, u4, u5, u6, u7, uo):
    cdims = (((0,), (0,)), ((), ()))

    def pack8(blocks):
        x_all = jnp.concatenate([b[...] for b in blocks], axis=0)
        y_a = lax.dot_general(x_all, ea_ref[...], cdims,
                              preferred_element_type=jnp.float32)
        y_b = lax.dot_general(x_all, eb_ref[...], cdims,
                              preferred_element_type=jnp.float32)
        return pltpu.pack_elementwise([y_a, y_b],
                                      packed_dtype=jnp.bfloat16)

    uo[...] = pack8((u0, u1, u2, u3, u4, u5, u6, u7))


def _repack(ut_t):
    def in_spec(p):
        return pl.BlockSpec(
            (_EMBED, _SEG), lambda g, p=p: (0, jnp.minimum(8 * g + p, _NBLK)))

    out_spec = pl.BlockSpec((_SEG, 128), lambda g: (g, 0))
    d_idx = jnp.arange(8 * _EMBED, dtype=jnp.int32)[:, None]
    l_idx = jnp.arange(128, dtype=jnp.int32)[None, :]
    tgt = _EMBED * (l_idx // 16) + 2 * (l_idx % 16)
    e_a = (d_idx == tgt).astype(jnp.float32)
    e_b = (d_idx == tgt + 1).astype(jnp.float32)
    const_spec = pl.BlockSpec((8 * _EMBED, 128), lambda g: (0, 0))
    return pl.pallas_call(
        _repack_body,
        grid=(_G,),
        in_specs=([const_spec, const_spec]
                  + [in_spec(p) for p in range(8)]),
        out_specs=out_spec,
        out_shape=jax.ShapeDtypeStruct((_PACKED_ROWS, 128), jnp.int32),
    )(e_a, e_b, *([ut_t] * 8))


def _sc_gather(packed, idx):
    mesh = plsc.VectorSubcoreMesh(core_axis_name="c", subcore_axis_name="s")

    @functools.partial(
        pl.kernel,
        mesh=mesh,
        compiler_params=pltpu.CompilerParams(needs_layout_passes=False),
        out_type=jax.ShapeDtypeStruct((_EMBED, _BATCH), jnp.float32),
        scratch_types=[
            pltpu.VMEM((_CHUNK,), jnp.int32),
            pltpu.VMEM((_CHUNK,), jnp.int32),
            pltpu.VMEM((_CHUNK, 128), jnp.int32),
            pltpu.VMEM((_EMBED, _CHUNK), jnp.float32),
            pltpu.SemaphoreType.DMA,
        ],
    )
    def k(up_hbm, uidx_hbm, uout_hbm,
          uidx_v, um_v, urows_v, uext_v, usem):
        wid = lax.axis_index("s") * _NC + lax.axis_index("c")
        iota16 = lax.iota(jnp.int32, 16)
        himask = jnp.full((16,), -65536, jnp.int32)

        def compute_m(iv, mv):
            def mbody(t, _):
                sl = pl.ds(16 * t, 16)
                v = iv[sl]
                mv[sl] = ((v >> (_SH + 3)) << _SH) | (v & (_SEG - 1))
                return _
            lax.fori_loop(0, _CHUNK // 16, mbody, 0)

        def extract(iv, rows, ext):
            def ebody(t, _):
                sl = pl.ds(16 * t, 16)
                lane0 = ((iv[sl] >> _SH) & 7) * 16
                jvec = iota16 + 16 * t
                for q in range(16):
                    w = plsc.load_gather(rows, [jvec, lane0 + q])
                    ext[2 * q, sl] = plsc.bitcast(w << 16, jnp.float32)
                    ext[2 * q + 1, sl] = plsc.bitcast(w & himask,
                                                      jnp.float32)
                return _
            lax.fori_loop(0, _CHUNK // 16, ebody, 0)

        for r in range(_BPW // _CHUNK):
            base = wid * _BPW + r * _CHUNK
            bsl = pl.ds(base, _CHUNK)
            pltpu.sync_copy(uidx_hbm.at[bsl], uidx_v)
            compute_m(uidx_v, um_v)
            pltpu.async_copy(up_hbm.at[um_v], urows_v, usem).wait()
            extract(uidx_v, urows_v, uext_v)
            pltpu.sync_copy(uext_v, uout_hbm.at[:, bsl])

    return k(packed, idx)


def _mlp_t_body(u_ref, g_ref, w1u_ref, w1g_ref, b1_ref, w2_ref, b2_ref,
                w3_ref, b3_ref, o_ref):
    cdims = (((0,), (0,)), ((), ()))
    h = lax.dot_general(w1u_ref[...], u_ref[...], cdims,
                        preferred_element_type=jnp.float32)
    h += lax.dot_general(w1g_ref[...], g_ref[...], cdims,
                         preferred_element_type=jnp.float32)
    h = jnp.maximum(h + b1_ref[...], 0.0)
    h = jnp.maximum(
        lax.dot_general(w2_ref[...], h, cdims,
                        preferred_element_type=jnp.float32) + b2_ref[...],
        0.0)
    o_ref[...] = (
        lax.dot_general(w3_ref[...], h, cdims,
                        preferred_element_type=jnp.float32) + b3_ref[...])


def _mlp_t(u_t, g_t, W1u, W1g, b1, W2, b2, W3, b3):
    bm = 2048
    h1 = W1u.shape[1]
    h2 = W2.shape[1]
    h3 = W3.shape[1]
    return pl.pallas_call(
        _mlp_t_body,
        grid=(_BATCH // bm,),
        in_specs=[
            pl.BlockSpec((_EMBED, bm), lambda i: (0, i)),
            pl.BlockSpec((_EMBED, bm), lambda i: (0, i)),
            pl.BlockSpec((_EMBED, h1), lambda i: (0, 0)),
            pl.BlockSpec((_EMBED, h1), lambda i: (0, 0)),
            pl.BlockSpec((h1, 1), lambda i: (0, 0)),
            pl.BlockSpec((h1, h2), lambda i: (0, 0)),
            pl.BlockSpec((h2, 1), lambda i: (0, 0)),
            pl.BlockSpec((h2, h3), lambda i: (0, 0)),
            pl.BlockSpec((h3, 1), lambda i: (0, 0)),
        ],
        out_specs=pl.BlockSpec((h3, bm), lambda i: (0, i)),
        out_shape=jax.ShapeDtypeStruct((h3, _BATCH), jnp.float32),
    )(u_t, g_t, W1u, W1g, b1.reshape(-1, 1), W2, b2.reshape(-1, 1), W3,
      b3.reshape(-1, 1))


def kernel(inputs, user_table, genre_table, W1, b1, W2, b2, W3, b3):
    u_idx = inputs[:, 0]
    g_idx = inputs[:, 1]
    u_packed = _repack(user_table.T)
    u_t = _sc_gather(u_packed, u_idx)
    g_packed = _repack(genre_table.T)
    g_t = _sc_gather(g_packed, g_idx)
    W1u = W1[:_EMBED]
    W1g = W1[_EMBED:]
    return _mlp_t(u_t, g_t, W1u, W1g, b1, W2, b2, W3, b3).T


# speedup vs baseline: 1.0668x; 1.0668x over previous
import functools

import jax
import jax.numpy as jnp
from jax import lax
from jax.experimental import pallas as pl
from jax.experimental.pallas import tpu as pltpu
from jax.experimental.pallas import tpu_sc as plsc

_EMBED = 32
_BATCH = 16384
_ROWS = 1000000
_NC = 2
_NS = 16
_NW = _NC * _NS
_BPW = _BATCH // _NW
_CHUNK = 256

_SEG = 4096
_SH = 12
_G = 31
_PACKED_ROWS = _G * _SEG
_NBLK = (_ROWS + _SEG - 1) // _SEG - 1


def _repack_body(ea_ref, eb_ref, u0, u1, u2, u3, u4, u5, u6, u7,
                 g0, g1, g2, g3, g4, g5, g6, g7, uo, go):
    cdims = (((0,), (0,)), ((), ()))

    def pack8(blocks):
        x_all = jnp.concatenate([b[...] for b in blocks], axis=0)
        y_a = lax.dot_general(x_all, ea_ref[...], cdims,
                              preferred_element_type=jnp.float32)
        y_b = lax.dot_general(x_all, eb_ref[...], cdims,
                              preferred_element_type=jnp.float32)
        return pltpu.pack_elementwise([y_a, y_b],
                                      packed_dtype=jnp.bfloat16)

    uo[...] = pack8((u0, u1, u2, u3, u4, u5, u6, u7))
    go[...] = pack8((g0, g1, g2, g3, g4, g5, g6, g7))


def _repack(ut_t, gt_t):
    def in_spec(p):
        return pl.BlockSpec(
            (_EMBED, _SEG), lambda g, p=p: (0, jnp.minimum(8 * g + p, _NBLK)))

    out_spec = pl.BlockSpec((_SEG, 128), lambda g: (g, 0))
    d_idx = jnp.arange(8 * _EMBED, dtype=jnp.int32)[:, None]
    l_idx = jnp.arange(128, dtype=jnp.int32)[None, :]
    tgt = _EMBED * (l_idx // 16) + 2 * (l_idx % 16)
    e_a = (d_idx == tgt).astype(jnp.float32)
    e_b = (d_idx == tgt + 1).astype(jnp.float32)
    const_spec = pl.BlockSpec((8 * _EMBED, 128), lambda g: (0, 0))
    return pl.pallas_call(
        _repack_body,
        grid=(_G,),
        in_specs=([const_spec, const_spec]
                  + [in_spec(p) for p in range(8)] * 2),
        out_specs=[out_spec, out_spec],
        out_shape=[
            jax.ShapeDtypeStruct((_PACKED_ROWS, 128), jnp.int32),
            jax.ShapeDtypeStruct((_PACKED_ROWS, 128), jnp.int32),
        ],
    )(e_a, e_b, *([ut_t] * 8), *([gt_t] * 8))


def _sc_gather(u_packed, g_packed, u_idx, g_idx):
    mesh = plsc.VectorSubcoreMesh(core_axis_name="c", subcore_axis_name="s")

    @functools.partial(
        pl.kernel,
        mesh=mesh,
        compiler_params=pltpu.CompilerParams(needs_layout_passes=False),
        out_type=[
            jax.ShapeDtypeStruct((_EMBED, _BATCH), jnp.float32),
            jax.ShapeDtypeStruct((_EMBED, _BATCH), jnp.float32),
        ],
        scratch_types=[
            pltpu.VMEM((_CHUNK,), jnp.int32),
            pltpu.VMEM((_CHUNK,), jnp.int32),
            pltpu.VMEM((_CHUNK, 128), jnp.int32),
            pltpu.VMEM((_EMBED, _CHUNK), jnp.float32),
            pltpu.VMEM((_CHUNK,), jnp.int32),
            pltpu.VMEM((_CHUNK,), jnp.int32),
            pltpu.VMEM((_CHUNK, 128), jnp.int32),
            pltpu.VMEM((_EMBED, _CHUNK), jnp.float32),
            pltpu.SemaphoreType.DMA,
            pltpu.SemaphoreType.DMA,
        ],
    )
    def k(up_hbm, gp_hbm, uidx_hbm, gidx_hbm, uout_hbm, gout_hbm,
          uidx_v, um_v, urows_v, uext_v, gidx_v, gm_v, grows_v, gext_v,
          usem, gsem):
        wid = lax.axis_index("s") * _NC + lax.axis_index("c")
        iota16 = lax.iota(jnp.int32, 16)
        himask = jnp.full((16,), -65536, jnp.int32)

        def compute_m(iv, mv):
            def mbody(t, _):
                sl = pl.ds(16 * t, 16)
                v = iv[sl]
                mv[sl] = ((v >> (_SH + 3)) << _SH) | (v & (_SEG - 1))
                return _
            lax.fori_loop(0, _CHUNK // 16, mbody, 0)

        def extract(iv, rows, ext):
            def ebody(t, _):
                sl = pl.ds(16 * t, 16)
                lane0 = ((iv[sl] >> _SH) & 7) * 16
                jvec = iota16 + 16 * t
                for q in range(16):
                    w = plsc.load_gather(rows, [jvec, lane0 + q])
                    ext[2 * q, sl] = plsc.bitcast(w << 16, jnp.float32)
                    ext[2 * q + 1, sl] = plsc.bitcast(w & himask,
                                                      jnp.float32)
                return _
            lax.fori_loop(0, _CHUNK // 16, ebody, 0)

        for r in range(_BPW // _CHUNK):
            base = wid * _BPW + r * _CHUNK
            bsl = pl.ds(base, _CHUNK)
            pltpu.sync_copy(uidx_hbm.at[bsl], uidx_v)
            pltpu.sync_copy(gidx_hbm.at[bsl], gidx_v)
            compute_m(uidx_v, um_v)
            compute_m(gidx_v, gm_v)
            ucp = pltpu.async_copy(up_hbm.at[um_v], urows_v, usem)
            gcp = pltpu.async_copy(gp_hbm.at[gm_v], grows_v, gsem)
            ucp.wait()
            extract(uidx_v, urows_v, uext_v)
            pltpu.sync_copy(uext_v, uout_hbm.at[:, bsl])
            gcp.wait()
            extract(gidx_v, grows_v, gext_v)
            pltpu.sync_copy(gext_v, gout_hbm.at[:, bsl])

    return k(u_packed, g_packed, u_idx, g_idx)


def _mlp_t_body(u_ref, g_ref, w1u_ref, w1g_ref, b1_ref, w2_ref, b2_ref,
                w3_ref, b3_ref, o_ref):
    cdims = (((0,), (0,)), ((), ()))
    h = lax.dot_general(w1u_ref[...], u_ref[...], cdims,
                        preferred_element_type=jnp.float32)
    h += lax.dot_general(w1g_ref[...], g_ref[...], cdims,
                         preferred_element_type=jnp.float32)
    h = jnp.maximum(h + b1_ref[...], 0.0)
    h = jnp.maximum(
        lax.dot_general(w2_ref[...], h, cdims,
                        preferred_element_type=jnp.float32) + b2_ref[...],
        0.0)
    o_ref[...] = (
        lax.dot_general(w3_ref[...], h, cdims,
                        preferred_element_type=jnp.float32) + b3_ref[...])


def _mlp_t(u_t, g_t, W1u, W1g, b1, W2, b2, W3, b3):
    bm = 2048
    h1 = W1u.shape[1]
    h2 = W2.shape[1]
    h3 = W3.shape[1]
    return pl.pallas_call(
        _mlp_t_body,
        grid=(_BATCH // bm,),
        in_specs=[
            pl.BlockSpec((_EMBED, bm), lambda i: (0, i)),
            pl.BlockSpec((_EMBED, bm), lambda i: (0, i)),
            pl.BlockSpec((_EMBED, h1), lambda i: (0, 0)),
            pl.BlockSpec((_EMBED, h1), lambda i: (0, 0)),
            pl.BlockSpec((h1, 1), lambda i: (0, 0)),
            pl.BlockSpec((h1, h2), lambda i: (0, 0)),
            pl.BlockSpec((h2, 1), lambda i: (0, 0)),
            pl.BlockSpec((h2, h3), lambda i: (0, 0)),
            pl.BlockSpec((h3, 1), lambda i: (0, 0)),
        ],
        out_specs=pl.BlockSpec((h3, bm), lambda i: (0, i)),
        out_shape=jax.ShapeDtypeStruct((h3, _BATCH), jnp.float32),
    )(u_t, g_t, W1u, W1g, b1.reshape(-1, 1), W2, b2.reshape(-1, 1), W3,
      b3.reshape(-1, 1))


def kernel(inputs, user_table, genre_table, W1, b1, W2, b2, W3, b3):
    u_idx = inputs[:, 0]
    g_idx = inputs[:, 1]
    u_packed, g_packed = _repack(user_table.T, genre_table.T)
    u_t, g_t = _sc_gather(u_packed, g_packed, u_idx, g_idx)
    W1u = W1[:_EMBED]
    W1g = W1[_EMBED:]
    return _mlp_t(u_t, g_t, W1u, W1g, b1, W2, b2, W3, b3).T


# speedup vs baseline: 1.0777x; 1.0103x over previous
import functools

import jax
import jax.numpy as jnp
from jax import lax
from jax.experimental import pallas as pl
from jax.experimental.pallas import tpu as pltpu
from jax.experimental.pallas import tpu_sc as plsc

_EMBED = 32
_BATCH = 16384
_ROWS = 1000000
_NC = 2
_NS = 16
_NW = _NC * _NS
_BPW = _BATCH // _NW
_CHUNK = 256

_SEG = 8192
_SH = 13
_G = 16
_PACKED_ROWS = _G * _SEG
_NBLK = (_ROWS + _SEG - 1) // _SEG - 1


def _repack_body(ea_ref, eb_ref, u0, u1, u2, u3, u4, u5, u6, u7,
                 g0, g1, g2, g3, g4, g5, g6, g7, uo, go):
    cdims = (((0,), (0,)), ((), ()))

    def pack8(blocks):
        x_all = jnp.concatenate([b[...] for b in blocks], axis=0)
        y_a = lax.dot_general(x_all, ea_ref[...], cdims,
                              preferred_element_type=jnp.float32)
        y_b = lax.dot_general(x_all, eb_ref[...], cdims,
                              preferred_element_type=jnp.float32)
        return pltpu.pack_elementwise([y_a, y_b],
                                      packed_dtype=jnp.bfloat16)

    uo[...] = pack8((u0, u1, u2, u3, u4, u5, u6, u7))
    go[...] = pack8((g0, g1, g2, g3, g4, g5, g6, g7))


def _repack(ut_t, gt_t):
    def in_spec(p):
        return pl.BlockSpec(
            (_EMBED, _SEG), lambda g, p=p: (0, jnp.minimum(8 * g + p, _NBLK)))

    out_spec = pl.BlockSpec((_SEG, 128), lambda g: (g, 0))
    d_idx = jnp.arange(8 * _EMBED, dtype=jnp.int32)[:, None]
    l_idx = jnp.arange(128, dtype=jnp.int32)[None, :]
    tgt = _EMBED * (l_idx // 16) + 2 * (l_idx % 16)
    e_a = (d_idx == tgt).astype(jnp.float32)
    e_b = (d_idx == tgt + 1).astype(jnp.float32)
    const_spec = pl.BlockSpec((8 * _EMBED, 128), lambda g: (0, 0))
    return pl.pallas_call(
        _repack_body,
        grid=(_G,),
        in_specs=([const_spec, const_spec]
                  + [in_spec(p) for p in range(8)] * 2),
        out_specs=[out_spec, out_spec],
        out_shape=[
            jax.ShapeDtypeStruct((_PACKED_ROWS, 128), jnp.int32),
            jax.ShapeDtypeStruct((_PACKED_ROWS, 128), jnp.int32),
        ],
    )(e_a, e_b, *([ut_t] * 8), *([gt_t] * 8))


def _sc_gather(u_packed, g_packed, u_idx, g_idx):
    mesh = plsc.VectorSubcoreMesh(core_axis_name="c", subcore_axis_name="s")

    @functools.partial(
        pl.kernel,
        mesh=mesh,
        compiler_params=pltpu.CompilerParams(needs_layout_passes=False),
        out_type=[
            jax.ShapeDtypeStruct((_EMBED, _BATCH), jnp.float32),
            jax.ShapeDtypeStruct((_EMBED, _BATCH), jnp.float32),
        ],
        scratch_types=[
            pltpu.VMEM((_CHUNK,), jnp.int32),
            pltpu.VMEM((_CHUNK,), jnp.int32),
            pltpu.VMEM((_CHUNK, 128), jnp.int32),
            pltpu.VMEM((_EMBED, _CHUNK), jnp.float32),
            pltpu.VMEM((_CHUNK,), jnp.int32),
            pltpu.VMEM((_CHUNK,), jnp.int32),
            pltpu.VMEM((_CHUNK, 128), jnp.int32),
            pltpu.VMEM((_EMBED, _CHUNK), jnp.float32),
            pltpu.SemaphoreType.DMA,
            pltpu.SemaphoreType.DMA,
        ],
    )
    def k(up_hbm, gp_hbm, uidx_hbm, gidx_hbm, uout_hbm, gout_hbm,
          uidx_v, um_v, urows_v, uext_v, gidx_v, gm_v, grows_v, gext_v,
          usem, gsem):
        wid = lax.axis_index("s") * _NC + lax.axis_index("c")
        iota16 = lax.iota(jnp.int32, 16)
        himask = jnp.full((16,), -65536, jnp.int32)

        def compute_m(iv, mv):
            def mbody(t, _):
                sl = pl.ds(16 * t, 16)
                v = iv[sl]
                mv[sl] = ((v >> (_SH + 3)) << _SH) | (v & (_SEG - 1))
                return _
            lax.fori_loop(0, _CHUNK // 16, mbody, 0)

        def extract(iv, rows, ext):
            def ebody(t, _):
                sl = pl.ds(16 * t, 16)
                lane0 = ((iv[sl] >> _SH) & 7) * 16
                jvec = iota16 + 16 * t
                for q in range(16):
                    w = plsc.load_gather(rows, [jvec, lane0 + q])
                    ext[2 * q, sl] = plsc.bitcast(w << 16, jnp.float32)
                    ext[2 * q + 1, sl] = plsc.bitcast(w & himask,
                                                      jnp.float32)
                return _
            lax.fori_loop(0, _CHUNK // 16, ebody, 0)

        for r in range(_BPW // _CHUNK):
            base = wid * _BPW + r * _CHUNK
            bsl = pl.ds(base, _CHUNK)
            pltpu.sync_copy(uidx_hbm.at[bsl], uidx_v)
            pltpu.sync_copy(gidx_hbm.at[bsl], gidx_v)
            compute_m(uidx_v, um_v)
            compute_m(gidx_v, gm_v)
            ucp = pltpu.async_copy(up_hbm.at[um_v], urows_v, usem)
            gcp = pltpu.async_copy(gp_hbm.at[gm_v], grows_v, gsem)
            ucp.wait()
            extract(uidx_v, urows_v, uext_v)
            pltpu.sync_copy(uext_v, uout_hbm.at[:, bsl])
            gcp.wait()
            extract(gidx_v, grows_v, gext_v)
            pltpu.sync_copy(gext_v, gout_hbm.at[:, bsl])

    return k(u_packed, g_packed, u_idx, g_idx)


def _mlp_t_body(u_ref, g_ref, w1u_ref, w1g_ref, b1_ref, w2_ref, b2_ref,
                w3_ref, b3_ref, o_ref):
    cdims = (((0,), (0,)), ((), ()))
    h = lax.dot_general(w1u_ref[...], u_ref[...], cdims,
                        preferred_element_type=jnp.float32)
    h += lax.dot_general(w1g_ref[...], g_ref[...], cdims,
                         preferred_element_type=jnp.float32)
    h = jnp.maximum(h + b1_ref[...], 0.0)
    h = jnp.maximum(
        lax.dot_general(w2_ref[...], h, cdims,
                        preferred_element_type=jnp.float32) + b2_ref[...],
        0.0)
    o_ref[...] = (
        lax.dot_general(w3_ref[...], h, cdims,
                        preferred_element_type=jnp.float32) + b3_ref[...])


def _mlp_t(u_t, g_t, W1u, W1g, b1, W2, b2, W3, b3):
    bm = 2048
    h1 = W1u.shape[1]
    h2 = W2.shape[1]
    h3 = W3.shape[1]
    return pl.pallas_call(
        _mlp_t_body,
        grid=(_BATCH // bm,),
        in_specs=[
            pl.BlockSpec((_EMBED, bm), lambda i: (0, i)),
            pl.BlockSpec((_EMBED, bm), lambda i: (0, i)),
            pl.BlockSpec((_EMBED, h1), lambda i: (0, 0)),
            pl.BlockSpec((_EMBED, h1), lambda i: (0, 0)),
            pl.BlockSpec((h1, 1), lambda i: (0, 0)),
            pl.BlockSpec((h1, h2), lambda i: (0, 0)),
            pl.BlockSpec((h2, 1), lambda i: (0, 0)),
            pl.BlockSpec((h2, h3), lambda i: (0, 0)),
            pl.BlockSpec((h3, 1), lambda i: (0, 0)),
        ],
        out_specs=pl.BlockSpec((h3, bm), lambda i: (0, i)),
        out_shape=jax.ShapeDtypeStruct((h3, _BATCH), jnp.float32),
    )(u_t, g_t, W1u, W1g, b1.reshape(-1, 1), W2, b2.reshape(-1, 1), W3,
      b3.reshape(-1, 1))


def kernel(inputs, user_table, genre_table, W1, b1, W2, b2, W3, b3):
    u_idx = inputs[:, 0]
    g_idx = inputs[:, 1]
    u_packed, g_packed = _repack(user_table.T, genre_table.T)
    u_t, g_t = _sc_gather(u_packed, g_packed, u_idx, g_idx)
    W1u = W1[:_EMBED]
    W1g = W1[_EMBED:]
    return _mlp_t(u_t, g_t, W1u, W1g, b1, W2, b2, W3, b3).T
